# Initial kernel scaffold; baseline (speedup 1.0000x reference)
import jax
import jax.numpy as jnp
from jax.experimental import pallas as pl


def kernel(z, edge_index, W, b):
    raise NotImplementedError("write your pallas kernel here")


# speedup vs baseline: 14.6383x; 14.6383x over previous
import functools

import jax
import jax.numpy as jnp
from jax import lax
from jax.experimental import pallas as pl
from jax.experimental.pallas import tpu as pltpu
from jax.experimental.pallas import tpu_sc as plsc

N_NODES = 10000
DIM = 64
NUM_CORES = 2
NUM_SUBCORES = 16
NUM_TILES = NUM_CORES * NUM_SUBCORES
CHUNK = 125
N_PAD = 10240
ROWS_PER_SUB = N_PAD // NUM_SUBCORES

_MESH = plsc.VectorSubcoreMesh(core_axis_name="c", subcore_axis_name="s")
_SC_PARAMS = pltpu.CompilerParams(use_tc_tiling_on_sc=False)


def _sc_degree(col2, ones_pay, zeros16):
    tot = col2.shape[0]
    per_tile = tot // NUM_TILES

    @functools.partial(
        pl.kernel, mesh=_MESH,
        out_type=jax.ShapeDtypeStruct((NUM_CORES, N_PAD, 16), jnp.float32),
        compiler_params=_SC_PARAMS,
        scratch_types=[
            pltpu.VMEM((per_tile, CHUNK), jnp.int32),
            pltpu.VMEM((CHUNK, 16), jnp.float32),
            pltpu.VMEM_SHARED((N_PAD, 16), jnp.float32),
        ])
    def k(col_hbm, ones_hbm, zeros_hbm, out_hbm, coli_v, ones_v, acc_sh):
        c = lax.axis_index("c")
        s = lax.axis_index("s")
        g = c * NUM_SUBCORES + s
        pltpu.sync_copy(zeros_hbm, acc_sh.at[pl.ds(s * ROWS_PER_SUB, ROWS_PER_SUB)])
        pltpu.sync_copy(ones_hbm, ones_v)
        pltpu.sync_copy(col_hbm.at[pl.ds(g * per_tile, per_tile)], coli_v)
        plsc.subcore_barrier()

        @pl.loop(0, per_tile)
        def _(i):
            pltpu.sync_copy(ones_v, acc_sh.at[coli_v.at[i]], add=True)

        plsc.subcore_barrier()
        pltpu.sync_copy(acc_sh.at[pl.ds(s * ROWS_PER_SUB, ROWS_PER_SUB)],
                        out_hbm.at[c, pl.ds(s * ROWS_PER_SUB, ROWS_PER_SUB)])

    return k(col2, ones_pay, zeros16)


def _sc_scatter(u, row2, col2, zeros64):
    tot = row2.shape[0]
    per_tile = tot // NUM_TILES

    @functools.partial(
        pl.kernel, mesh=_MESH,
        out_type=jax.ShapeDtypeStruct((NUM_CORES, N_PAD, DIM), jnp.float32),
        compiler_params=_SC_PARAMS,
        scratch_types=[
            pltpu.VMEM((per_tile, CHUNK), jnp.int32),
            pltpu.VMEM((per_tile, CHUNK), jnp.int32),
            pltpu.VMEM((CHUNK, DIM), jnp.float32),
            pltpu.VMEM((CHUNK, DIM), jnp.float32),
            pltpu.VMEM_SHARED((N_PAD, DIM), jnp.float32),
            pltpu.SemaphoreType.DMA,
            pltpu.SemaphoreType.DMA,
        ])
    def k(u_hbm, row_hbm, col_hbm, zeros_hbm, out_hbm,
          rowi_v, coli_v, buf_a, buf_b, acc_sh, sem_a, sem_b):
        c = lax.axis_index("c")
        s = lax.axis_index("s")
        g = c * NUM_SUBCORES + s
        pltpu.sync_copy(zeros_hbm, acc_sh.at[pl.ds(s * ROWS_PER_SUB, ROWS_PER_SUB)])
        pltpu.sync_copy(row_hbm.at[pl.ds(g * per_tile, per_tile)], rowi_v)
        pltpu.sync_copy(col_hbm.at[pl.ds(g * per_tile, per_tile)], coli_v)
        plsc.subcore_barrier()

        pltpu.async_copy(u_hbm.at[rowi_v.at[0]], buf_a, sem_a)

        @pl.loop(0, per_tile, step=2)
        def _(i):
            pltpu.make_async_copy(u_hbm.at[rowi_v.at[i]], buf_a, sem_a).wait()
            pltpu.async_copy(u_hbm.at[rowi_v.at[i + 1]], buf_b, sem_b)
            pltpu.sync_copy(buf_a, acc_sh.at[coli_v.at[i]], add=True)
            pltpu.make_async_copy(u_hbm.at[rowi_v.at[i + 1]], buf_b, sem_b).wait()

            @pl.when(i + 2 < per_tile)
            def _():
                pltpu.async_copy(u_hbm.at[rowi_v.at[i + 2]], buf_a, sem_a)

            pltpu.sync_copy(buf_b, acc_sh.at[coli_v.at[i + 1]], add=True)

        plsc.subcore_barrier()
        pltpu.sync_copy(acc_sh.at[pl.ds(s * ROWS_PER_SUB, ROWS_PER_SUB)],
                        out_hbm.at[c, pl.ds(s * ROWS_PER_SUB, ROWS_PER_SUB)])

    return k(u, row2, col2, zeros64)


def _tc_xw(z, W):
    bm = 2000

    def body(z_ref, w_ref, o_ref):
        o_ref[...] = jax.lax.dot(z_ref[...], w_ref[...],
                                 precision=lax.Precision.HIGHEST,
                                 preferred_element_type=jnp.float32)

    return pl.pallas_call(
        body,
        grid=(N_NODES // bm,),
        in_specs=[pl.BlockSpec((bm, DIM), lambda i: (i, 0)),
                  pl.BlockSpec((DIM, DIM), lambda i: (0, 0))],
        out_specs=pl.BlockSpec((bm, DIM), lambda i: (i, 0)),
        out_shape=jax.ShapeDtypeStruct((N_NODES, DIM), jnp.float32),
    )(z, W)


def _tc_gram(h_pad):
    bm, bn = 1024, 2048
    n_pad = h_pad.shape[0]
    gi = (N_NODES + bm - 1) // bm
    gj = (N_NODES + bn - 1) // bn

    def body(h_ref, o_ref):
        i = pl.program_id(0)
        j = pl.program_id(1)
        a = h_ref[pl.ds(i * bm, bm), :]
        b = h_ref[pl.ds(j * bn, bn), :]
        o_ref[...] = jax.lax.dot_general(
            a, b, (((1,), (1,)), ((), ())),
            precision=lax.Precision.HIGHEST,
            preferred_element_type=jnp.float32)

    return pl.pallas_call(
        body,
        grid=(gi, gj),
        in_specs=[pl.BlockSpec((n_pad, DIM), lambda i, j: (0, 0))],
        out_specs=pl.BlockSpec((bm, bn), lambda i, j: (i, j)),
        out_shape=jax.ShapeDtypeStruct((N_NODES, N_NODES), jnp.float32),
    )(h_pad)


def kernel(z, edge_index, W, b):
    row = edge_index[0].astype(jnp.int32).reshape(-1, CHUNK)
    col = edge_index[1].astype(jnp.int32).reshape(-1, CHUNK)

    ones_pay = jnp.ones((CHUNK, 16), jnp.float32)
    zeros16 = jnp.zeros((ROWS_PER_SUB, 16), jnp.float32)
    zeros64 = jnp.zeros((ROWS_PER_SUB, DIM), jnp.float32)

    hist = _sc_degree(col, ones_pay, zeros16)
    xw = _tc_xw(z, W)

    deg = 1.0 + hist[0, :N_NODES, 0] + hist[1, :N_NODES, 0]
    dinv = lax.rsqrt(deg)
    u = dinv[:, None] * xw

    part = _sc_scatter(u, row, col, zeros64)
    acc = part[0, :N_NODES] + part[1, :N_NODES]

    h = jax.nn.relu(dinv[:, None] * acc + dinv[:, None] ** 2 * xw + b)

    h_pad = jnp.zeros((N_PAD, DIM), jnp.float32).at[:N_NODES].set(h)
    return _tc_gram(h_pad)


# speedup vs baseline: 24.4116x; 1.6676x over previous
import functools

import jax
import jax.numpy as jnp
from jax import lax
from jax.experimental import pallas as pl
from jax.experimental.pallas import tpu as pltpu
from jax.experimental.pallas import tpu_sc as plsc

N_NODES = 10000
DIM = 64
NUM_CORES = 2
NUM_SUBCORES = 16
NUM_TILES = NUM_CORES * NUM_SUBCORES
CHUNK = 125
N_PAD = 10240
ROWS_PER_SUB = N_PAD // NUM_SUBCORES

_MESH = plsc.VectorSubcoreMesh(core_axis_name="c", subcore_axis_name="s")
_SC_PARAMS = pltpu.CompilerParams(use_tc_tiling_on_sc=False)


def _sc_degree(col2, ones_pay, zeros16):
    tot = col2.shape[0]
    per_tile = tot // NUM_TILES

    @functools.partial(
        pl.kernel, mesh=_MESH,
        out_type=jax.ShapeDtypeStruct((NUM_CORES, N_PAD, 16), jnp.float32),
        compiler_params=_SC_PARAMS,
        scratch_types=[
            pltpu.VMEM((per_tile, CHUNK), jnp.int32),
            pltpu.VMEM((CHUNK, 16), jnp.float32),
            pltpu.VMEM_SHARED((N_PAD, 16), jnp.float32),
        ])
    def k(col_hbm, ones_hbm, zeros_hbm, out_hbm, coli_v, ones_v, acc_sh):
        c = lax.axis_index("c")
        s = lax.axis_index("s")
        g = c * NUM_SUBCORES + s
        pltpu.sync_copy(zeros_hbm, acc_sh.at[pl.ds(s * ROWS_PER_SUB, ROWS_PER_SUB)])
        pltpu.sync_copy(ones_hbm, ones_v)
        pltpu.sync_copy(col_hbm.at[pl.ds(g * per_tile, per_tile)], coli_v)
        plsc.subcore_barrier()

        @pl.loop(0, per_tile)
        def _(i):
            pltpu.sync_copy(ones_v, acc_sh.at[coli_v.at[i]], add=True)

        plsc.subcore_barrier()
        pltpu.sync_copy(acc_sh.at[pl.ds(s * ROWS_PER_SUB, ROWS_PER_SUB)],
                        out_hbm.at[c, pl.ds(s * ROWS_PER_SUB, ROWS_PER_SUB)])

    return k(col2, ones_pay, zeros16)


def _sc_scatter(u, row2, col2, zeros64):
    tot = row2.shape[0]
    per_tile = tot // NUM_TILES

    @functools.partial(
        pl.kernel, mesh=_MESH,
        out_type=jax.ShapeDtypeStruct((NUM_CORES, N_PAD, DIM), jnp.float32),
        compiler_params=_SC_PARAMS,
        scratch_types=[
            pltpu.VMEM((per_tile, CHUNK), jnp.int32),
            pltpu.VMEM((per_tile, CHUNK), jnp.int32),
            pltpu.VMEM((CHUNK, DIM), jnp.float32),
            pltpu.VMEM((CHUNK, DIM), jnp.float32),
            pltpu.VMEM_SHARED((N_PAD, DIM), jnp.float32),
            pltpu.SemaphoreType.DMA,
            pltpu.SemaphoreType.DMA,
        ])
    def k(u_hbm, row_hbm, col_hbm, zeros_hbm, out_hbm,
          rowi_v, coli_v, buf_a, buf_b, acc_sh, sem_a, sem_b):
        c = lax.axis_index("c")
        s = lax.axis_index("s")
        g = c * NUM_SUBCORES + s
        pltpu.sync_copy(zeros_hbm, acc_sh.at[pl.ds(s * ROWS_PER_SUB, ROWS_PER_SUB)])
        pltpu.sync_copy(row_hbm.at[pl.ds(g * per_tile, per_tile)], rowi_v)
        pltpu.sync_copy(col_hbm.at[pl.ds(g * per_tile, per_tile)], coli_v)
        plsc.subcore_barrier()

        pltpu.async_copy(u_hbm.at[rowi_v.at[0]], buf_a, sem_a)

        @pl.loop(0, per_tile, step=2)
        def _(i):
            pltpu.make_async_copy(u_hbm.at[rowi_v.at[i]], buf_a, sem_a).wait()
            pltpu.async_copy(u_hbm.at[rowi_v.at[i + 1]], buf_b, sem_b)
            pltpu.sync_copy(buf_a, acc_sh.at[coli_v.at[i]], add=True)
            pltpu.make_async_copy(u_hbm.at[rowi_v.at[i + 1]], buf_b, sem_b).wait()

            @pl.when(i + 2 < per_tile)
            def _():
                pltpu.async_copy(u_hbm.at[rowi_v.at[i + 2]], buf_a, sem_a)

            pltpu.sync_copy(buf_b, acc_sh.at[coli_v.at[i + 1]], add=True)

        plsc.subcore_barrier()
        pltpu.sync_copy(acc_sh.at[pl.ds(s * ROWS_PER_SUB, ROWS_PER_SUB)],
                        out_hbm.at[c, pl.ds(s * ROWS_PER_SUB, ROWS_PER_SUB)])

    return k(u, row2, col2, zeros64)


def _tc_xw(z, W):
    bm = 2000

    def body(z_ref, w_ref, o_ref):
        o_ref[...] = jax.lax.dot(z_ref[...], w_ref[...],
                                 precision=lax.Precision.HIGHEST,
                                 preferred_element_type=jnp.float32)

    return pl.pallas_call(
        body,
        grid=(N_NODES // bm,),
        in_specs=[pl.BlockSpec((bm, DIM), lambda i: (i, 0)),
                  pl.BlockSpec((DIM, DIM), lambda i: (0, 0))],
        out_specs=pl.BlockSpec((bm, DIM), lambda i: (i, 0)),
        out_shape=jax.ShapeDtypeStruct((N_NODES, DIM), jnp.float32),
    )(z, W)


def _tc_gram(h_pad):
    bm, bn = 1024, 2048
    n_pad = h_pad.shape[0]
    gi = (N_NODES + bm - 1) // bm
    gj = (N_NODES + bn - 1) // bn

    def body(h_ref, o_ref):
        i = pl.program_id(0)
        j = pl.program_id(1)
        a = h_ref[pl.ds(i * bm, bm), :]
        b = h_ref[pl.ds(j * bn, bn), :]
        o_ref[...] = jax.lax.dot_general(
            a, b, (((1,), (1,)), ((), ())),
            preferred_element_type=jnp.float32)

    return pl.pallas_call(
        body,
        grid=(gi, gj),
        in_specs=[pl.BlockSpec((n_pad, DIM), lambda i, j: (0, 0))],
        out_specs=pl.BlockSpec((bm, bn), lambda i, j: (i, j)),
        out_shape=jax.ShapeDtypeStruct((N_NODES, N_NODES), jnp.float32),
    )(h_pad)


def kernel(z, edge_index, W, b):
    row = edge_index[0].astype(jnp.int32).reshape(-1, CHUNK)
    col = edge_index[1].astype(jnp.int32).reshape(-1, CHUNK)

    ones_pay = jnp.ones((CHUNK, 16), jnp.float32)
    zeros16 = jnp.zeros((ROWS_PER_SUB, 16), jnp.float32)
    zeros64 = jnp.zeros((ROWS_PER_SUB, DIM), jnp.float32)

    hist = _sc_degree(col, ones_pay, zeros16)
    xw = _tc_xw(z, W)

    deg = 1.0 + hist[0, :N_NODES, 0] + hist[1, :N_NODES, 0]
    dinv = lax.rsqrt(deg)
    u = dinv[:, None] * xw

    part = _sc_scatter(u, row, col, zeros64)
    acc = part[0, :N_NODES] + part[1, :N_NODES]

    h = jax.nn.relu(dinv[:, None] * acc + dinv[:, None] ** 2 * xw + b)

    h_pad = jnp.zeros((N_PAD, DIM), jnp.float32).at[:N_NODES].set(h)
    return _tc_gram(h_pad)
